# Initial kernel scaffold; baseline (speedup 1.0000x reference)
import jax
import jax.numpy as jnp
from jax.experimental import pallas as pl


def kernel(x, edge_index, edge_attr, W1q, b1q, W1k, b1k, W1v, b1v, W1e, b1e, W1s, b1s, Wbq, bbq, Wbk, bbk, Wbv, bbv, Wbe, bbe, Wbs, bbs, Wl, bl):
    raise NotImplementedError("write your pallas kernel here")


# speedup vs baseline: 12.6798x; 12.6798x over previous
import functools
import math

import jax
import jax.numpy as jnp
from jax import lax
from jax.experimental import pallas as pl
from jax.experimental.pallas import tpu as pltpu
from jax.experimental.pallas import tpu_sc as plsc

N = 10000
E = 320000
ND = 128
ED = 16
H = 32
NL = 8
ACC_W = 48
NW = 32
EW = E // NW
C = 80
NCH = EW // C
RPS = N // 16
INV_SQRT_H = 1.0 / math.sqrt(float(H))
F32 = jnp.float32


def _sc_body(q_hbm, k_hbm, v_hbm, e_hbm, src_hbm, dst_hbm, z_hbm, out_hbm,
             dsti, srci, qr, kr, vr, er, st, acc, s0, s1, s2, s3):
    c = lax.axis_index("c")
    s = lax.axis_index("s")
    wid = s * 2 + c

    pltpu.sync_copy(z_hbm, acc.at[pl.ds(s * RPS, RPS)])

    plsc.subcore_barrier()

    base0 = wid * EW

    def chunk_body(i, carry):
        base = base0 + i * C
        pltpu.sync_copy(dst_hbm.at[pl.ds(base, C)], dsti)
        pltpu.sync_copy(src_hbm.at[pl.ds(base, C)], srci)
        cq = pltpu.async_copy(q_hbm.at[dsti], qr, s0)
        ck = pltpu.async_copy(k_hbm.at[srci], kr, s1)
        cv = pltpu.async_copy(v_hbm.at[srci], vr, s2)
        ce = pltpu.async_copy(e_hbm.at[pl.ds(base, C)], er, s3)
        cq.wait()
        ck.wait()
        cv.wait()
        ce.wait()
        for r in range(C):
            ea = er[r, pl.ds(0, 16)]
            eb = er[r, pl.ds(16, 16)]
            ka = kr[r, pl.ds(0, 16)] + ea
            kb = kr[r, pl.ds(16, 16)] + eb
            t = qr[r, pl.ds(0, 16)] * ka + qr[r, pl.ds(16, 16)] * kb
            s_val = jnp.sum(t) * INV_SQRT_H
            w = jnp.exp(jnp.full((16,), s_val, F32))
            st[r, pl.ds(0, 16)] = (vr[r, pl.ds(0, 16)] + ea) * w
            st[r, pl.ds(16, 16)] = (vr[r, pl.ds(16, 16)] + eb) * w
            st[r, pl.ds(32, 16)] = w
        pltpu.sync_copy(st, acc.at[dsti], add=True)
        return carry

    lax.fori_loop(0, NCH, chunk_body, 0)

    plsc.subcore_barrier()

    pltpu.sync_copy(acc.at[pl.ds(s * RPS, RPS)], out_hbm.at[c, s])


_sc_edge_pass = functools.partial(
    pl.kernel,
    mesh=plsc.VectorSubcoreMesh(core_axis_name="c", subcore_axis_name="s"),
    out_type=jax.ShapeDtypeStruct((2, 16, RPS, ACC_W), F32),
    compiler_params=pltpu.CompilerParams(
        needs_layout_passes=False, use_tc_tiling_on_sc=False),
    scratch_types=[
        pltpu.VMEM((C,), jnp.int32),
        pltpu.VMEM((C,), jnp.int32),
        pltpu.VMEM((C, H), F32),
        pltpu.VMEM((C, H), F32),
        pltpu.VMEM((C, H), F32),
        pltpu.VMEM((C, H), F32),
        pltpu.VMEM((C, ACC_W), F32),
        pltpu.VMEM_SHARED((N, ACC_W), F32),
        pltpu.SemaphoreType.DMA,
        pltpu.SemaphoreType.DMA,
        pltpu.SemaphoreType.DMA,
        pltpu.SemaphoreType.DMA,
    ],
)(_sc_body)


EC = 4000


def _e_body(ea_ref, We_ref, be_ref, out_ref):
    ea = jnp.log(ea_ref[...] + 1.0)
    We = We_ref[...]
    be = be_ref[...]
    for l in range(NL):
        out_ref[l] = jnp.dot(ea, We[l], preferred_element_type=F32) + be[l]


_e_kernel = pl.pallas_call(
    _e_body,
    grid=(E // EC,),
    in_specs=[
        pl.BlockSpec((EC, ED), lambda i: (i, 0)),
        pl.BlockSpec((NL, ED, H), lambda i: (0, 0, 0)),
        pl.BlockSpec((NL, H), lambda i: (0, 0)),
    ],
    out_specs=pl.BlockSpec((NL, EC, H), lambda i: (0, i, 0)),
    out_shape=jax.ShapeDtypeStruct((NL, E, H), F32),
)


def _node0_body(x_ref, wq, bq, wk, bk, wv, bv, ws, bs, q_o, k_o, v_o, sk_o):
    h0 = jnp.log(x_ref[...] + 1.0)
    q_o[...] = jnp.dot(h0, wq[...], preferred_element_type=F32) + bq[...]
    k_o[...] = jnp.dot(h0, wk[...], preferred_element_type=F32) + bk[...]
    v_o[...] = jnp.dot(h0, wv[...], preferred_element_type=F32) + bv[...]
    sk_o[...] = jnp.dot(h0, ws[...], preferred_element_type=F32) + bs[...]


_node0_kernel = pl.pallas_call(
    _node0_body,
    out_shape=[jax.ShapeDtypeStruct((N, H), F32)] * 4,
)


def _block_body(acc_ref, sk_ref, h_ref, wq, bq, wk, bk, wv, bv, ws, bs,
                h_o, q_o, k_o, v_o, sk_o):
    a = acc_ref[...]
    num = a[0, :, :H] + a[1, :, :H]
    den = a[0, :, H:H + 1] + a[1, :, H:H + 1]
    msg = num / (den + 1e-16)
    h = h_ref[...] + msg + sk_ref[...]
    h_o[...] = h
    mean = jnp.mean(h, axis=-1, keepdims=True)
    var = jnp.sum((h - mean) ** 2, axis=-1, keepdims=True) * (1.0 / (H - 1))
    h2 = jnp.maximum(h * jax.lax.rsqrt(var), 0.0)
    q_o[...] = jnp.dot(h2, wq[...], preferred_element_type=F32) + bq[...]
    k_o[...] = jnp.dot(h2, wk[...], preferred_element_type=F32) + bk[...]
    v_o[...] = jnp.dot(h2, wv[...], preferred_element_type=F32) + bv[...]
    sk_o[...] = jnp.dot(h2, ws[...], preferred_element_type=F32) + bs[...]


_block_kernel = pl.pallas_call(
    _block_body,
    out_shape=[jax.ShapeDtypeStruct((N, H), F32)] * 5,
)


def _final_body(acc_ref, sk_ref, h_ref, wl, bl, out_ref):
    a = acc_ref[...]
    num = a[0, :, :H] + a[1, :, :H]
    den = a[0, :, H:H + 1] + a[1, :, H:H + 1]
    h = h_ref[...] + num / (den + 1e-16) + sk_ref[...]
    h = h * (1.0 / math.sqrt(float(NL - 1)))
    out_ref[...] = jnp.dot(h, wl[...], preferred_element_type=F32) + bl[...]


_final_kernel = pl.pallas_call(
    _final_body,
    out_shape=jax.ShapeDtypeStruct((N, 1), F32),
)


def kernel(x, edge_index, edge_attr, W1q, b1q, W1k, b1k, W1v, b1v, W1e, b1e,
           W1s, b1s, Wbq, bbq, Wbk, bbk, Wbv, bbv, Wbe, bbe, Wbs, bbs, Wl, bl):
    src = edge_index[0]
    dst = edge_index[1]
    We_all = jnp.concatenate([W1e[None], Wbe], axis=0)
    be_all = jnp.concatenate([b1e[None], bbe], axis=0)
    e_all = _e_kernel(edge_attr, We_all, be_all)
    q, k, v, sk = _node0_kernel(x, W1q, b1q, W1k, b1k, W1v, b1v, W1s, b1s)
    h = jnp.zeros((N, H), F32)
    zeros48 = jnp.zeros((RPS, ACC_W), F32)
    for l in range(NL):
        acc = _sc_edge_pass(q, k, v, e_all[l], src, dst, zeros48)
        acc = acc.reshape(2, N, ACC_W)
        if l < NL - 1:
            h, q, k, v, sk = _block_kernel(
                acc, sk, h, Wbq[l], bbq[l], Wbk[l], bbk[l],
                Wbv[l], bbv[l], Wbs[l], bbs[l])
        else:
            out = _final_kernel(acc, sk, h, Wl, bl)
    return out


# speedup vs baseline: 16.8112x; 1.3258x over previous
import functools
import math

import jax
import jax.numpy as jnp
from jax import lax
from jax.experimental import pallas as pl
from jax.experimental.pallas import tpu as pltpu
from jax.experimental.pallas import tpu_sc as plsc

N = 10000
E = 320000
ND = 128
ED = 16
H = 32
NL = 8
ACC_W = 48
NW = 32
EW = E // NW
C = 80
NCH = EW // C
RPS = N // 16
INV_SQRT_H = 1.0 / math.sqrt(float(H))
F32 = jnp.float32


def _sc_body(q_hbm, k_hbm, v_hbm, e_hbm, src_hbm, dst_hbm, z_hbm, out_hbm,
             dsti, srci, qr, kr, vr, er, st, acc, sg):
    c = lax.axis_index("c")
    s = lax.axis_index("s")
    wid = s * 2 + c

    pltpu.sync_copy(z_hbm, acc.at[pl.ds(s * RPS, RPS)])

    pltpu.sync_copy(dst_hbm.at[wid], dsti)
    pltpu.sync_copy(src_hbm.at[wid], srci)

    plsc.subcore_barrier()

    base0 = wid * EW

    def fire(i, b):
        pltpu.async_copy(q_hbm.at[dsti.at[i]], qr.at[b], sg.at[b])
        pltpu.async_copy(k_hbm.at[srci.at[i]], kr.at[b], sg.at[b])
        pltpu.async_copy(v_hbm.at[srci.at[i]], vr.at[b], sg.at[b])
        pltpu.async_copy(e_hbm.at[pl.ds(base0 + i * C, C)], er.at[b],
                         sg.at[b])

    def wait_gathers(b):
        pltpu.make_async_copy(q_hbm.at[pl.ds(0, C)], qr.at[b], sg.at[b]).wait()
        pltpu.make_async_copy(k_hbm.at[pl.ds(0, C)], kr.at[b], sg.at[b]).wait()
        pltpu.make_async_copy(v_hbm.at[pl.ds(0, C)], vr.at[b], sg.at[b]).wait()
        pltpu.make_async_copy(e_hbm.at[pl.ds(0, C)], er.at[b], sg.at[b]).wait()

    def step(i, b, fire_next):
        if fire_next:
            fire(i + 1, 1 - b)
        wait_gathers(b)
        for r in range(C):
            ea = er[b, r, pl.ds(0, 16)]
            eb = er[b, r, pl.ds(16, 16)]
            ka = kr[b, r, pl.ds(0, 16)] + ea
            kb = kr[b, r, pl.ds(16, 16)] + eb
            t = qr[b, r, pl.ds(0, 16)] * ka + qr[b, r, pl.ds(16, 16)] * kb
            s_val = jnp.sum(t) * INV_SQRT_H
            w = jnp.exp(jnp.full((16,), s_val, F32))
            st[r, pl.ds(0, 16)] = (vr[b, r, pl.ds(0, 16)] + ea) * w
            st[r, pl.ds(16, 16)] = (vr[b, r, pl.ds(16, 16)] + eb) * w
            st[r, pl.ds(32, 16)] = w
        pltpu.sync_copy(st, acc.at[dsti.at[i]], add=True)

    fire(0, 0)

    def outer_body(io, carry):
        step(io * 2, 0, True)
        step(io * 2 + 1, 1, True)
        return carry

    lax.fori_loop(0, NCH // 2, outer_body, 0)
    step(NCH - 1, 0, False)

    plsc.subcore_barrier()

    pltpu.sync_copy(acc.at[pl.ds(s * RPS, RPS)], out_hbm.at[c, s])


_sc_edge_pass = functools.partial(
    pl.kernel,
    mesh=plsc.VectorSubcoreMesh(core_axis_name="c", subcore_axis_name="s"),
    out_type=jax.ShapeDtypeStruct((2, 16, RPS, ACC_W), F32),
    compiler_params=pltpu.CompilerParams(
        needs_layout_passes=False, use_tc_tiling_on_sc=False),
    scratch_types=[
        pltpu.VMEM((NCH, C), jnp.int32),
        pltpu.VMEM((NCH, C), jnp.int32),
        pltpu.VMEM((2, C, H), F32),
        pltpu.VMEM((2, C, H), F32),
        pltpu.VMEM((2, C, H), F32),
        pltpu.VMEM((2, C, H), F32),
        pltpu.VMEM((C, ACC_W), F32),
        pltpu.VMEM_SHARED((N, ACC_W), F32),
        pltpu.SemaphoreType.DMA((2,)),
    ],
)(_sc_body)


EC = 4000


def _e_body(ea_ref, We_ref, be_ref, out_ref):
    ea = jnp.log(ea_ref[...] + 1.0)
    big = jnp.dot(ea, We_ref[...], preferred_element_type=F32)
    be = be_ref[...]
    for l in range(NL):
        out_ref[l] = big[:, l * H:(l + 1) * H] + be[l]


_e_kernel = pl.pallas_call(
    _e_body,
    grid=(E // EC,),
    in_specs=[
        pl.BlockSpec((EC, ED), lambda i: (i, 0)),
        pl.BlockSpec((ED, NL * H), lambda i: (0, 0)),
        pl.BlockSpec((NL, H), lambda i: (0, 0)),
    ],
    out_specs=pl.BlockSpec((NL, EC, H), lambda i: (0, i, 0)),
    out_shape=jax.ShapeDtypeStruct((NL, E, H), F32),
)


def _node0_body(x_ref, wq, bq, wk, bk, wv, bv, ws, bs, q_o, k_o, v_o, sk_o):
    h0 = jnp.log(x_ref[...] + 1.0)
    q_o[...] = jnp.dot(h0, wq[...], preferred_element_type=F32) + bq[...]
    k_o[...] = jnp.dot(h0, wk[...], preferred_element_type=F32) + bk[...]
    v_o[...] = jnp.dot(h0, wv[...], preferred_element_type=F32) + bv[...]
    sk_o[...] = jnp.dot(h0, ws[...], preferred_element_type=F32) + bs[...]


_node0_kernel = pl.pallas_call(
    _node0_body,
    out_shape=[jax.ShapeDtypeStruct((N, H), F32)] * 4,
)


def _block_body(acc_ref, sk_ref, h_ref, wq, bq, wk, bk, wv, bv, ws, bs,
                h_o, q_o, k_o, v_o, sk_o):
    a = acc_ref[...]
    num = a[0, :, :H] + a[1, :, :H]
    den = a[0, :, H:H + 1] + a[1, :, H:H + 1]
    msg = num / (den + 1e-16)
    h = h_ref[...] + msg + sk_ref[...]
    h_o[...] = h
    mean = jnp.mean(h, axis=-1, keepdims=True)
    var = jnp.sum((h - mean) ** 2, axis=-1, keepdims=True) * (1.0 / (H - 1))
    h2 = jnp.maximum(h * jax.lax.rsqrt(var), 0.0)
    q_o[...] = jnp.dot(h2, wq[...], preferred_element_type=F32) + bq[...]
    k_o[...] = jnp.dot(h2, wk[...], preferred_element_type=F32) + bk[...]
    v_o[...] = jnp.dot(h2, wv[...], preferred_element_type=F32) + bv[...]
    sk_o[...] = jnp.dot(h2, ws[...], preferred_element_type=F32) + bs[...]


_block_kernel = pl.pallas_call(
    _block_body,
    out_shape=[jax.ShapeDtypeStruct((N, H), F32)] * 5,
)


def _final_body(acc_ref, sk_ref, h_ref, wl, bl, out_ref):
    a = acc_ref[...]
    num = a[0, :, :H] + a[1, :, :H]
    den = a[0, :, H:H + 1] + a[1, :, H:H + 1]
    h = h_ref[...] + num / (den + 1e-16) + sk_ref[...]
    h = h * (1.0 / math.sqrt(float(NL - 1)))
    out_ref[...] = jnp.dot(h, wl[...], preferred_element_type=F32) + bl[...]


_final_kernel = pl.pallas_call(
    _final_body,
    out_shape=jax.ShapeDtypeStruct((N, 1), F32),
)


def kernel(x, edge_index, edge_attr, W1q, b1q, W1k, b1k, W1v, b1v, W1e, b1e,
           W1s, b1s, Wbq, bbq, Wbk, bbk, Wbv, bbv, Wbe, bbe, Wbs, bbs, Wl, bl):
    src = edge_index[0].reshape(NW, NCH, C)
    dst = edge_index[1].reshape(NW, NCH, C)
    We_all = jnp.concatenate([W1e[None], Wbe], axis=0)
    We_all = jnp.transpose(We_all, (1, 0, 2)).reshape(ED, NL * H)
    be_all = jnp.concatenate([b1e[None], bbe], axis=0)
    e_all = _e_kernel(edge_attr, We_all, be_all)
    q, k, v, sk = _node0_kernel(x, W1q, b1q, W1k, b1k, W1v, b1v, W1s, b1s)
    h = jnp.zeros((N, H), F32)
    zeros48 = jnp.zeros((RPS, ACC_W), F32)
    for l in range(NL):
        acc = _sc_edge_pass(q, k, v, e_all[l], src, dst, zeros48)
        acc = acc.reshape(2, N, ACC_W)
        if l < NL - 1:
            h, q, k, v, sk = _block_kernel(
                acc, sk, h, Wbq[l], bbq[l], Wbk[l], bbk[l],
                Wbv[l], bbv[l], Wbs[l], bbs[l])
        else:
            out = _final_kernel(acc, sk, h, Wl, bl)
    return out


# speedup vs baseline: 27.6363x; 1.6439x over previous
import functools
import math

import jax
import jax.numpy as jnp
import numpy as np
from jax import lax
from jax.experimental import pallas as pl
from jax.experimental.pallas import tpu as pltpu
from jax.experimental.pallas import tpu_sc as plsc

N = 10000
E = 320000
ND = 128
ED = 16
H = 32
NL = 8
ACC_W = 48
NW = 32
EW = E // NW
C = 80
NCH = EW // C
RPS = N // 16
INV_SQRT_H = 1.0 / math.sqrt(float(H))
F32 = jnp.float32
BF16 = jnp.bfloat16
PERM = tuple(range(0, H, 2)) + tuple(range(1, H, 2))


CR = C // 4
ER = E // 4


def _sc_body(q_hbm, k_hbm, v_hbm, e_hbm, src_hbm, dst_hbm, z_hbm, lofs_hbm,
             out_hbm, dsti, srci, qr, kr, vr, er, st, acc, lofs_sm, sg):
    c = lax.axis_index("c")
    s = lax.axis_index("s")
    wid = s * 2 + c

    pltpu.sync_copy(z_hbm, acc.at[pl.ds(s * RPS, RPS)])

    pltpu.sync_copy(dst_hbm.at[wid], dsti)
    pltpu.sync_copy(src_hbm.at[wid], srci)
    pltpu.sync_copy(lofs_hbm, lofs_sm)

    plsc.subcore_barrier()

    ebase = lofs_sm[...][0] + wid * (EW // 4)

    def fire(i, b):
        pltpu.async_copy(q_hbm.at[dsti.at[i]], qr.at[b], sg.at[b])
        pltpu.async_copy(k_hbm.at[srci.at[i]], kr.at[b], sg.at[b])
        pltpu.async_copy(v_hbm.at[srci.at[i]], vr.at[b], sg.at[b])
        pltpu.async_copy(e_hbm.at[pl.ds(ebase + i * CR, CR)], er.at[b],
                         sg.at[b])

    def wait_gathers(b):
        pltpu.make_async_copy(q_hbm.at[pl.ds(0, C)], qr.at[b], sg.at[b]).wait()
        pltpu.make_async_copy(k_hbm.at[pl.ds(0, C)], kr.at[b], sg.at[b]).wait()
        pltpu.make_async_copy(v_hbm.at[pl.ds(0, C)], vr.at[b], sg.at[b]).wait()
        pltpu.make_async_copy(e_hbm.at[pl.ds(0, CR)], er.at[b],
                              sg.at[b]).wait()

    def step(i, b, fire_next):
        if fire_next:
            fire(i + 1, 1 - b)
        wait_gathers(b)
        for r in range(C):
            fmt = plsc.PackFormat.INTERLEAVED
            ea, eb = plsc.unpack(
                er[b, r // 4, pl.ds((r % 4) * H, H)], format=fmt)
            ka, kb = plsc.unpack(kr[b, r, pl.ds(0, H)], format=fmt)
            qa, qb = plsc.unpack(qr[b, r, pl.ds(0, H)], format=fmt)
            va, vb = plsc.unpack(vr[b, r, pl.ds(0, H)], format=fmt)
            t = qa * (ka + ea) + qb * (kb + eb)
            s_val = jnp.sum(t) * INV_SQRT_H
            w = jnp.exp(jnp.full((16,), s_val, F32))
            st[r, pl.ds(0, 16)] = (va + ea) * w
            st[r, pl.ds(16, 16)] = (vb + eb) * w
            st[r, pl.ds(32, 16)] = w
        pltpu.sync_copy(st, acc.at[dsti.at[i]], add=True)

    fire(0, 0)

    def outer_body(io, carry):
        step(io * 2, 0, True)
        step(io * 2 + 1, 1, True)
        return carry

    lax.fori_loop(0, NCH // 2, outer_body, 0)
    step(NCH - 1, 0, False)

    plsc.subcore_barrier()

    pltpu.sync_copy(acc.at[pl.ds(s * RPS, RPS)], out_hbm.at[c, s])


_sc_edge_pass = functools.partial(
    pl.kernel,
    mesh=plsc.VectorSubcoreMesh(core_axis_name="c", subcore_axis_name="s"),
    out_type=jax.ShapeDtypeStruct((2, 16, RPS, ACC_W), F32),
    compiler_params=pltpu.CompilerParams(
        needs_layout_passes=False, use_tc_tiling_on_sc=False),
    scratch_types=[
        pltpu.VMEM((NCH, C), jnp.int32),
        pltpu.VMEM((NCH, C), jnp.int32),
        pltpu.VMEM((2, C, H), BF16),
        pltpu.VMEM((2, C, H), BF16),
        pltpu.VMEM((2, C, H), BF16),
        pltpu.VMEM((2, CR, 4 * H), BF16),
        pltpu.VMEM((C, ACC_W), F32),
        pltpu.VMEM_SHARED((N, ACC_W), F32),
        pltpu.VMEM((16,), jnp.int32),
        pltpu.SemaphoreType.DMA((2,)),
    ],
)(_sc_body)


EC4 = 2000


def _e_body(ea_ref, We_ref, be_ref, out_ref):
    ea = jnp.log(ea_ref[...] + 1.0)
    big = jnp.dot(ea, We_ref[...], preferred_element_type=F32)
    be = be_ref[...]
    for l in range(NL):
        out_ref[l] = (big[:, l * 4 * H:(l + 1) * 4 * H] + be[l]).astype(BF16)


_e_kernel = pl.pallas_call(
    _e_body,
    grid=(ER // EC4,),
    in_specs=[
        pl.BlockSpec((EC4, 4 * ED), lambda i: (i, 0)),
        pl.BlockSpec((4 * ED, NL * 4 * H), lambda i: (0, 0)),
        pl.BlockSpec((NL, 4 * H), lambda i: (0, 0)),
    ],
    out_specs=pl.BlockSpec((NL, EC4, 4 * H), lambda i: (0, i, 0)),
    out_shape=jax.ShapeDtypeStruct((NL, ER, 4 * H), BF16),
)


def _node0_body(x_ref, wq, bq, wk, bk, wv, bv, ws, bs, q_o, k_o, v_o, sk_o):
    h0 = jnp.log(x_ref[...] + 1.0)
    q_o[...] = (jnp.dot(h0, wq[...], preferred_element_type=F32)
                + bq[...]).astype(BF16)
    k_o[...] = (jnp.dot(h0, wk[...], preferred_element_type=F32)
                + bk[...]).astype(BF16)
    v_o[...] = (jnp.dot(h0, wv[...], preferred_element_type=F32)
                + bv[...]).astype(BF16)
    sk_o[...] = jnp.dot(h0, ws[...], preferred_element_type=F32) + bs[...]


_node0_kernel = pl.pallas_call(
    _node0_body,
    out_shape=[jax.ShapeDtypeStruct((N, H), BF16)] * 3
    + [jax.ShapeDtypeStruct((N, H), F32)],
)


def _block_body(acc_ref, sk_ref, h_ref, wq, bq, wk, bk, wv, bv, ws, bs,
                h_o, q_o, k_o, v_o, sk_o):
    a = acc_ref[...]
    num = a[0, :, :H] + a[1, :, :H]
    den = a[0, :, H:H + 1] + a[1, :, H:H + 1]
    msg = num / (den + 1e-16)
    h = h_ref[...] + msg + sk_ref[...]
    h_o[...] = h
    mean = jnp.mean(h, axis=-1, keepdims=True)
    var = jnp.sum((h - mean) ** 2, axis=-1, keepdims=True) * (1.0 / (H - 1))
    h2 = jnp.maximum(h * jax.lax.rsqrt(var), 0.0)
    q_o[...] = (jnp.dot(h2, wq[...], preferred_element_type=F32)
                + bq[...]).astype(BF16)
    k_o[...] = (jnp.dot(h2, wk[...], preferred_element_type=F32)
                + bk[...]).astype(BF16)
    v_o[...] = (jnp.dot(h2, wv[...], preferred_element_type=F32)
                + bv[...]).astype(BF16)
    sk_o[...] = jnp.dot(h2, ws[...], preferred_element_type=F32) + bs[...]


_block_kernel = pl.pallas_call(
    _block_body,
    out_shape=[jax.ShapeDtypeStruct((N, H), F32)]
    + [jax.ShapeDtypeStruct((N, H), BF16)] * 3
    + [jax.ShapeDtypeStruct((N, H), F32)],
)


def _final_body(acc_ref, sk_ref, h_ref, wl, bl, out_ref):
    a = acc_ref[...]
    num = a[0, :, :H] + a[1, :, :H]
    den = a[0, :, H:H + 1] + a[1, :, H:H + 1]
    h = h_ref[...] + num / (den + 1e-16) + sk_ref[...]
    h = h * (1.0 / math.sqrt(float(NL - 1)))
    out_ref[...] = jnp.dot(h, wl[...], preferred_element_type=F32) + bl[...]


_final_kernel = pl.pallas_call(
    _final_body,
    out_shape=jax.ShapeDtypeStruct((N, 1), F32),
)


def kernel(x, edge_index, edge_attr, W1q, b1q, W1k, b1k, W1v, b1v, W1e, b1e,
           W1s, b1s, Wbq, bbq, Wbk, bbk, Wbv, bbv, Wbe, bbe, Wbs, bbs, Wl, bl):
    src = edge_index[0].reshape(NW, NCH, C)
    dst = edge_index[1].reshape(NW, NCH, C)
    eye4 = jnp.eye(4, dtype=F32)
    We_list = [W1e] + [Wbe[i] for i in range(NL - 1)]
    be_list = [b1e] + [bbe[i] for i in range(NL - 1)]
    We_all = jnp.concatenate([jnp.kron(eye4, W) for W in We_list], axis=1)
    be_all = jnp.stack([jnp.tile(b, 4) for b in be_list], axis=0)
    ea4 = edge_attr.reshape(ER, 4 * ED)
    e_all = _e_kernel(ea4, We_all, be_all).reshape(NL * ER, 4 * H)
    p = np.asarray(PERM)
    W1s_p = W1s[:, p]
    b1s_p = b1s[p]
    Wbq_p = Wbq[:, p, :]
    Wbk_p = Wbk[:, p, :]
    Wbv_p = Wbv[:, p, :]
    Wbs_p = Wbs[:, p, :][:, :, p]
    bbs_p = bbs[:, p]
    Wl_p = Wl[p, :]
    q, k, v, sk = _node0_kernel(x, W1q, b1q, W1k, b1k, W1v, b1v,
                                W1s_p, b1s_p)
    h = jnp.zeros((N, H), F32)
    zeros48 = jnp.zeros((RPS, ACC_W), F32)
    for l in range(NL):
        lofs = jnp.full((16,), l * ER, jnp.int32)
        acc = _sc_edge_pass(q, k, v, e_all, src, dst, zeros48, lofs)
        acc = acc.reshape(2, N, ACC_W)
        if l < NL - 1:
            h, q, k, v, sk = _block_kernel(
                acc, sk, h, Wbq_p[l], bbq[l], Wbk_p[l], bbk[l],
                Wbv_p[l], bbv[l], Wbs_p[l], bbs_p[l])
        else:
            out = _final_kernel(acc, sk, h, Wl_p, bl)
    return out


# speedup vs baseline: 29.5608x; 1.0696x over previous
import functools
import math

import jax
import jax.numpy as jnp
import numpy as np
from jax import lax
from jax.experimental import pallas as pl
from jax.experimental.pallas import tpu as pltpu
from jax.experimental.pallas import tpu_sc as plsc

N = 10000
E = 320000
ND = 128
ED = 16
H = 32
NL = 8
ACC_W = 48
NW = 32
EW = E // NW
C = 80
NCH = EW // C
RPS = N // 16
INV_SQRT_H = 1.0 / math.sqrt(float(H))
F32 = jnp.float32
BF16 = jnp.bfloat16
PERM = tuple(range(0, H, 2)) + tuple(range(1, H, 2))


CR = C // 4
ER = E // 4


def _sc_body(qkv_hbm, e_hbm, ei_hbm, z_hbm, lofs_hbm,
             out_hbm, dsti, srci, sni, s2ni, qr, kr, vr, er, st, acc,
             lofs_sm, sg, ss):
    c = lax.axis_index("c")
    s = lax.axis_index("s")
    wid = s * 2 + c

    pltpu.sync_copy(z_hbm, acc.at[pl.ds(s * RPS, RPS)])

    pltpu.sync_copy(ei_hbm.at[1, wid], dsti)
    pltpu.sync_copy(ei_hbm.at[0, wid], srci)
    pltpu.sync_copy(lofs_hbm, lofs_sm)

    def offs_body(i, carry):
        for j in range(C // 16):
            vsrc = srci[i, pl.ds(j * 16, 16)]
            sni[i, pl.ds(j * 16, 16)] = vsrc + N
            s2ni[i, pl.ds(j * 16, 16)] = vsrc + 2 * N
        return carry

    lax.fori_loop(0, NCH, offs_body, 0)

    plsc.subcore_barrier()

    lidx = lofs_sm[...][0]
    ebase = wid * (EW // 4)

    def fire(i, b):
        pltpu.async_copy(qkv_hbm.at[dsti.at[i]], qr.at[b], sg.at[b])
        pltpu.async_copy(qkv_hbm.at[sni.at[i]], kr.at[b], sg.at[b])
        pltpu.async_copy(qkv_hbm.at[s2ni.at[i]], vr.at[b], sg.at[b])
        pltpu.async_copy(e_hbm.at[lidx, pl.ds(ebase + i * CR, CR)],
                         er.at[b], sg.at[b])

    def wait_gathers(b):
        pltpu.make_async_copy(qkv_hbm.at[pl.ds(0, C)], qr.at[b],
                              sg.at[b]).wait()
        pltpu.make_async_copy(qkv_hbm.at[pl.ds(0, C)], kr.at[b],
                              sg.at[b]).wait()
        pltpu.make_async_copy(qkv_hbm.at[pl.ds(0, C)], vr.at[b],
                              sg.at[b]).wait()
        pltpu.make_async_copy(e_hbm.at[0, pl.ds(0, CR)], er.at[b],
                              sg.at[b]).wait()

    def step(i, b, fire_next):
        if fire_next:
            fire(i + 1, 1 - b)
        wait_gathers(b)

        @pl.when(i >= 2)
        def _():
            pltpu.make_async_copy(st.at[b], acc.at[dsti.at[i]],
                                  ss.at[b]).wait()

        for r in range(C):
            fmt = plsc.PackFormat.INTERLEAVED
            ea, eb = plsc.unpack(
                er[b, r // 4, pl.ds((r % 4) * H, H)], format=fmt)
            ka, kb = plsc.unpack(kr[b, r, pl.ds(0, H)], format=fmt)
            qa, qb = plsc.unpack(qr[b, r, pl.ds(0, H)], format=fmt)
            va, vb = plsc.unpack(vr[b, r, pl.ds(0, H)], format=fmt)
            t = qa * (ka + ea) + qb * (kb + eb)
            s_val = jnp.sum(t) * INV_SQRT_H
            w = jnp.exp(jnp.full((16,), s_val, F32))
            st[b, r, pl.ds(0, 16)] = (va + ea) * w
            st[b, r, pl.ds(16, 16)] = (vb + eb) * w
            st[b, r, pl.ds(32, 16)] = w
        pltpu.async_copy(st.at[b], acc.at[dsti.at[i]], ss.at[b], add=True)

    fire(0, 0)

    def outer_body(io, carry):
        step(io * 2, 0, True)
        step(io * 2 + 1, 1, True)
        return carry

    lax.fori_loop(0, NCH // 2, outer_body, 0)
    step(NCH - 1, 0, False)

    pltpu.make_async_copy(st.at[0], acc.at[dsti.at[0]], ss.at[0]).wait()
    pltpu.make_async_copy(st.at[1], acc.at[dsti.at[0]], ss.at[1]).wait()

    plsc.subcore_barrier()

    pltpu.sync_copy(acc.at[pl.ds(s * RPS, RPS)], out_hbm.at[c, s])


_sc_edge_pass = functools.partial(
    pl.kernel,
    mesh=plsc.VectorSubcoreMesh(core_axis_name="c", subcore_axis_name="s"),
    out_type=jax.ShapeDtypeStruct((2, 16, RPS, ACC_W), F32),
    compiler_params=pltpu.CompilerParams(
        needs_layout_passes=False, use_tc_tiling_on_sc=False),
    scratch_types=[
        pltpu.VMEM((NCH, C), jnp.int32),
        pltpu.VMEM((NCH, C), jnp.int32),
        pltpu.VMEM((NCH, C), jnp.int32),
        pltpu.VMEM((NCH, C), jnp.int32),
        pltpu.VMEM((2, C, H), BF16),
        pltpu.VMEM((2, C, H), BF16),
        pltpu.VMEM((2, C, H), BF16),
        pltpu.VMEM((2, CR, 4 * H), BF16),
        pltpu.VMEM((2, C, ACC_W), F32),
        pltpu.VMEM_SHARED((N, ACC_W), F32),
        pltpu.VMEM((16,), jnp.int32),
        pltpu.SemaphoreType.DMA((2,)),
        pltpu.SemaphoreType.DMA((2,)),
    ],
)(_sc_body)


EC4 = 2000


def _e_body(ea_ref, We_ref, be_ref, out_ref):
    ea = jnp.log(ea_ref[...] + 1.0)
    big = jnp.dot(ea, We_ref[...], preferred_element_type=F32)
    be = be_ref[...]
    for l in range(NL):
        out_ref[l] = (big[:, l * 4 * H:(l + 1) * 4 * H] + be[l]).astype(BF16)


_e_kernel = pl.pallas_call(
    _e_body,
    grid=(ER // EC4,),
    in_specs=[
        pl.BlockSpec((EC4, 4 * ED), lambda i: (i, 0)),
        pl.BlockSpec((4 * ED, NL * 4 * H), lambda i: (0, 0)),
        pl.BlockSpec((NL, 4 * H), lambda i: (0, 0)),
    ],
    out_specs=pl.BlockSpec((NL, EC4, 4 * H), lambda i: (0, i, 0)),
    out_shape=jax.ShapeDtypeStruct((NL, ER, 4 * H), BF16),
)


def _node0_body(x_ref, wq, bq, wk, bk, wv, bv, ws, bs, qkv_o, sk_o):
    h0 = jnp.log(x_ref[...] + 1.0)
    qkv_o[0] = (jnp.dot(h0, wq[...], preferred_element_type=F32)
                + bq[...]).astype(BF16)
    qkv_o[1] = (jnp.dot(h0, wk[...], preferred_element_type=F32)
                + bk[...]).astype(BF16)
    qkv_o[2] = (jnp.dot(h0, wv[...], preferred_element_type=F32)
                + bv[...]).astype(BF16)
    sk_o[...] = jnp.dot(h0, ws[...], preferred_element_type=F32) + bs[...]


_node0_kernel = pl.pallas_call(
    _node0_body,
    out_shape=[jax.ShapeDtypeStruct((3, N, H), BF16),
               jax.ShapeDtypeStruct((N, H), F32)],
)


def _block_body(acc_ref, sk_ref, h_ref, wq, bq, wk, bk, wv, bv, ws, bs,
                h_o, qkv_o, sk_o):
    a = acc_ref[...]
    num = a[0, :, :H] + a[1, :, :H]
    den = a[0, :, H:H + 1] + a[1, :, H:H + 1]
    msg = num / (den + 1e-16)
    h = h_ref[...] + msg + sk_ref[...]
    h_o[...] = h
    mean = jnp.mean(h, axis=-1, keepdims=True)
    var = jnp.sum((h - mean) ** 2, axis=-1, keepdims=True) * (1.0 / (H - 1))
    h2 = jnp.maximum(h * jax.lax.rsqrt(var), 0.0)
    qkv_o[0] = (jnp.dot(h2, wq[...], preferred_element_type=F32)
                + bq[...]).astype(BF16)
    qkv_o[1] = (jnp.dot(h2, wk[...], preferred_element_type=F32)
                + bk[...]).astype(BF16)
    qkv_o[2] = (jnp.dot(h2, wv[...], preferred_element_type=F32)
                + bv[...]).astype(BF16)
    sk_o[...] = jnp.dot(h2, ws[...], preferred_element_type=F32) + bs[...]


_block_kernel = pl.pallas_call(
    _block_body,
    out_shape=[jax.ShapeDtypeStruct((N, H), F32),
               jax.ShapeDtypeStruct((3, N, H), BF16),
               jax.ShapeDtypeStruct((N, H), F32)],
)


def _final_body(acc_ref, sk_ref, h_ref, wl, bl, out_ref):
    a = acc_ref[...]
    num = a[0, :, :H] + a[1, :, :H]
    den = a[0, :, H:H + 1] + a[1, :, H:H + 1]
    h = h_ref[...] + num / (den + 1e-16) + sk_ref[...]
    h = h * (1.0 / math.sqrt(float(NL - 1)))
    out_ref[...] = jnp.dot(h, wl[...], preferred_element_type=F32) + bl[...]


_final_kernel = pl.pallas_call(
    _final_body,
    out_shape=jax.ShapeDtypeStruct((N, 1), F32),
)


def kernel(x, edge_index, edge_attr, W1q, b1q, W1k, b1k, W1v, b1v, W1e, b1e,
           W1s, b1s, Wbq, bbq, Wbk, bbk, Wbv, bbv, Wbe, bbe, Wbs, bbs, Wl, bl):
    ei = edge_index.reshape(2, NW, NCH, C)
    eye4 = jnp.eye(4, dtype=F32)
    We_list = [W1e] + [Wbe[i] for i in range(NL - 1)]
    be_list = [b1e] + [bbe[i] for i in range(NL - 1)]
    We_all = jnp.concatenate([jnp.kron(eye4, W) for W in We_list], axis=1)
    be_all = jnp.stack([jnp.tile(b, 4) for b in be_list], axis=0)
    ea4 = edge_attr.reshape(ER, 4 * ED)
    e_all = _e_kernel(ea4, We_all, be_all)
    p = np.asarray(PERM)
    W1s_p = W1s[:, p]
    b1s_p = b1s[p]
    Wbq_p = Wbq[:, p, :]
    Wbk_p = Wbk[:, p, :]
    Wbv_p = Wbv[:, p, :]
    Wbs_p = Wbs[:, p, :][:, :, p]
    bbs_p = bbs[:, p]
    Wl_p = Wl[p, :]
    qkv, sk = _node0_kernel(x, W1q, b1q, W1k, b1k, W1v, b1v, W1s_p, b1s_p)
    h = jnp.zeros((N, H), F32)
    zeros_acc = jnp.zeros((RPS, ACC_W), F32)
    for l in range(NL):
        lofs = jnp.full((16,), l, jnp.int32)
        acc = _sc_edge_pass(qkv.reshape(3 * N, H), e_all, ei,
                            zeros_acc, lofs)
        acc = acc.reshape(2, N, ACC_W)
        if l < NL - 1:
            h, qkv, sk = _block_kernel(
                acc, sk, h, Wbq_p[l], bbq[l], Wbk_p[l], bbk[l],
                Wbv_p[l], bbv[l], Wbs_p[l], bbs_p[l])
        else:
            out = _final_kernel(acc, sk, h, Wl_p, bl)
    return out
